# Initial kernel scaffold; baseline (speedup 1.0000x reference)
import jax
import jax.numpy as jnp
from jax.experimental import pallas as pl


def kernel(x, edge_index, edge_weight, W, b):
    raise NotImplementedError("write your pallas kernel here")


# speedup vs baseline: 5.5252x; 5.5252x over previous
import jax
import jax.numpy as jnp
from jax import lax
from jax.experimental import pallas as pl
from jax.experimental.pallas import tpu as pltpu
from jax.experimental.pallas import tpu_sc as plsc

N_NODES = 10000
IN_CH = 128
OUT_CH = 128

NC = 2
NS = 16
L = 16

NPAD = 10240
ROWS_PER_TILE = NPAD // NS
CHUNK = 128
CPW = 88
GRP = 8
NGRP = CPW // GRP
NW = NC * NS
EPAD = NW * CPW * CHUNK

_sc_mesh = plsc.VectorSubcoreMesh(
    core_axis_name="c", subcore_axis_name="s", num_cores=NC, num_subcores=NS
)


def _mm_body(x_ref, w_ref, o_ref):
    o_ref[...] = jnp.dot(x_ref[...], w_ref[...], preferred_element_type=jnp.float32)


def _matmul(xp, W):
    blk = 512
    return pl.pallas_call(
        _mm_body,
        grid=(NPAD // blk,),
        in_specs=[
            pl.BlockSpec((blk, IN_CH), lambda i: (i, 0)),
            pl.BlockSpec((IN_CH, OUT_CH), lambda i: (0, 0)),
        ],
        out_specs=pl.BlockSpec((blk, OUT_CH), lambda i: (i, 0)),
        out_shape=jax.ShapeDtypeStruct((NPAD, OUT_CH), jnp.float32),
    )(xp, W)


def _deg_body(dst_hbm, ew_hbm, deg_hbm, dstv, ewv, degsh, zb, sem):
    c = lax.axis_index("c")
    s = lax.axis_index("s")
    wid = c * NS + s
    base = wid * CPW
    pltpu.sync_copy(dst_hbm.at[pl.ds(base, CPW)], dstv)
    pltpu.sync_copy(ew_hbm.at[pl.ds(base, CPW)], ewv)

    def zloop(i, carry):
        zb[pl.ds(i * L, L)] = jnp.zeros((L,), jnp.float32)
        return carry

    lax.fori_loop(0, ROWS_PER_TILE // L, zloop, 0)
    pltpu.sync_copy(zb, degsh.at[pl.ds(s * ROWS_PER_TILE, ROWS_PER_TILE)])
    plsc.subcore_barrier()

    def sloop(j, carry):
        pltpu.async_copy(ewv.at[j], degsh.at[dstv.at[j]], sem, add=True)
        return carry

    lax.fori_loop(0, CPW, sloop, 0)

    def dloop(j, carry):
        pltpu.make_async_copy(ewv.at[j], degsh.at[dstv.at[j]], sem).wait()
        return carry

    lax.fori_loop(0, CPW, dloop, 0)
    plsc.subcore_barrier()
    pltpu.sync_copy(degsh.at[pl.ds(s * ROWS_PER_TILE, ROWS_PER_TILE)], zb)
    pltpu.sync_copy(zb, deg_hbm.at[pl.ds(c * NPAD + s * ROWS_PER_TILE, ROWS_PER_TILE)])


def _deg(dst2, ew2):
    return pl.kernel(
        _deg_body,
        out_type=jax.ShapeDtypeStruct((NC * NPAD,), jnp.float32),
        mesh=_sc_mesh,
        scratch_types=[
            pltpu.VMEM((CPW, CHUNK), jnp.int32),
            pltpu.VMEM((CPW, CHUNK), jnp.float32),
            pltpu.VMEM_SHARED((NPAD,), jnp.float32),
            pltpu.VMEM((ROWS_PER_TILE,), jnp.float32),
            pltpu.SemaphoreType.DMA,
        ],
    )(dst2, ew2)


def _dis_body(deg_ref, dis_ref):
    d = deg_ref[0] + deg_ref[1]
    dis_ref[...] = jnp.where(d > 0, lax.rsqrt(jnp.where(d > 0, d, 1.0)), 0.0)


def _dis(deg_part):
    return pl.pallas_call(
        _dis_body,
        out_shape=jax.ShapeDtypeStruct((NPAD // 128, 128), jnp.float32),
    )(deg_part)


def _scale_body(h_ref, dis_ref, o_ref):
    o_ref[...] = h_ref[...] * dis_ref[...]


def _scale(h, dis2):
    blk = 512
    return pl.pallas_call(
        _scale_body,
        grid=(NPAD // blk,),
        in_specs=[
            pl.BlockSpec((blk, OUT_CH), lambda i: (i, 0)),
            pl.BlockSpec((blk, 1), lambda i: (i, 0)),
        ],
        out_specs=pl.BlockSpec((blk, OUT_CH), lambda i: (i, 0)),
        out_shape=jax.ShapeDtypeStruct((NPAD, OUT_CH), jnp.float32),
    )(h, dis2)


def _msg_body(src_hbm, dst_hbm, ew_hbm, hs_hbm, out_hbm,
              srcv, dstv, ewv, rowb, accsh, gsem, ssem):
    c = lax.axis_index("c")
    s = lax.axis_index("s")
    wid = c * NS + s
    base = wid * CPW

    def zr(r, carry):
        for q in range(OUT_CH // L):
            rowb[r, pl.ds(q * L, L)] = jnp.zeros((L,), jnp.float32)
        return carry

    lax.fori_loop(0, CHUNK, zr, 0)
    for k in range(ROWS_PER_TILE // CHUNK):
        pltpu.sync_copy(rowb, accsh.at[pl.ds(s * ROWS_PER_TILE + k * CHUNK, CHUNK)])
    plsc.subcore_barrier()

    def gloop(g, carry):
        gbase = base + g * GRP
        pltpu.sync_copy(src_hbm.at[pl.ds(gbase, GRP)], srcv)
        pltpu.sync_copy(dst_hbm.at[pl.ds(gbase, GRP)], dstv)
        pltpu.sync_copy(ew_hbm.at[pl.ds(gbase, GRP)], ewv)

        def eloop(j, carry1):
            pltpu.async_copy(hs_hbm.at[srcv.at[j]], rowb, gsem).wait()

            def scale(rg, carry2):
                n16 = ewv[j, pl.ds(rg * L, L)]
                for i in range(L):
                    nb = n16.at[jnp.full((L,), i, jnp.int32)].get(
                        mode="promise_in_bounds")
                    r = rg * L + i
                    for q in range(OUT_CH // L):
                        sl = pl.ds(q * L, L)
                        rowb[r, sl] = rowb[r, sl] * nb
                return carry2

            lax.fori_loop(0, CHUNK // L, scale, 0)
            pltpu.async_copy(rowb, accsh.at[dstv.at[j]], ssem, add=True).wait()
            return carry1

        lax.fori_loop(0, GRP, eloop, 0)
        return carry

    lax.fori_loop(0, NGRP, gloop, 0)
    plsc.subcore_barrier()
    for k in range(ROWS_PER_TILE // CHUNK):
        off = s * ROWS_PER_TILE + k * CHUNK
        pltpu.sync_copy(accsh.at[pl.ds(off, CHUNK)], rowb)
        pltpu.sync_copy(rowb, out_hbm.at[c, pl.ds(off, CHUNK)])


def _msg(src2, dst2, ew2, hs):
    return pl.kernel(
        _msg_body,
        out_type=jax.ShapeDtypeStruct((NC, NPAD, OUT_CH), jnp.float32),
        mesh=_sc_mesh,
        scratch_types=[
            pltpu.VMEM((GRP, CHUNK), jnp.int32),
            pltpu.VMEM((GRP, CHUNK), jnp.int32),
            pltpu.VMEM((GRP, CHUNK), jnp.float32),
            pltpu.VMEM((CHUNK, OUT_CH), jnp.float32),
            pltpu.VMEM_SHARED((NPAD, OUT_CH), jnp.float32),
            pltpu.SemaphoreType.DMA,
            pltpu.SemaphoreType.DMA,
        ],
    )(src2, dst2, ew2, hs)


def _comb_body(p0_ref, p1_ref, dis_ref, b_ref, o_ref):
    o_ref[...] = (p0_ref[...] + p1_ref[...]) * dis_ref[...] + b_ref[...]


def _combine(p0, p1, dis2, b2):
    blk = 512
    return pl.pallas_call(
        _comb_body,
        grid=(NPAD // blk,),
        in_specs=[
            pl.BlockSpec((blk, OUT_CH), lambda i: (i, 0)),
            pl.BlockSpec((blk, OUT_CH), lambda i: (i, 0)),
            pl.BlockSpec((blk, 1), lambda i: (i, 0)),
            pl.BlockSpec((1, OUT_CH), lambda i: (0, 0)),
        ],
        out_specs=pl.BlockSpec((blk, OUT_CH), lambda i: (i, 0)),
        out_shape=jax.ShapeDtypeStruct((NPAD, OUT_CH), jnp.float32),
    )(p0, p1, dis2, b2)


def kernel(x, edge_index, edge_weight, W, b):
    E = edge_index.shape[1]
    src = edge_index[0].astype(jnp.int32)
    dst = edge_index[1].astype(jnp.int32)
    ew = edge_weight.astype(jnp.float32)

    loop = jnp.arange(NPAD, dtype=jnp.int32)
    n_fill = EPAD - E - NPAD
    src_f = jnp.concatenate([src, loop, jnp.zeros((n_fill,), jnp.int32)])
    dst_f = jnp.concatenate([dst, loop, jnp.zeros((n_fill,), jnp.int32)])
    ew_f = jnp.concatenate(
        [ew, jnp.ones((NPAD,), jnp.float32), jnp.zeros((n_fill,), jnp.float32)]
    )
    src2 = src_f.reshape(EPAD // CHUNK, CHUNK)
    dst2 = dst_f.reshape(EPAD // CHUNK, CHUNK)
    ew2 = ew_f.reshape(EPAD // CHUNK, CHUNK)

    xp = jnp.pad(x.astype(jnp.float32), ((0, NPAD - N_NODES), (0, 0)))
    h = _matmul(xp, W.astype(jnp.float32))

    deg_part = _deg(dst2, ew2)
    dis = _dis(deg_part.reshape(NC, NPAD // 128, 128)).reshape(NPAD, 1)

    hs = _scale(h, dis)
    parts = _msg(src2, dst2, ew2, hs)
    out = _combine(parts[0], parts[1], dis,
                   b.astype(jnp.float32).reshape(1, OUT_CH))
    return out[:N_NODES]


# speedup vs baseline: 22.2095x; 4.0197x over previous
import jax
import jax.numpy as jnp
from jax import lax
from jax.experimental import pallas as pl
from jax.experimental.pallas import tpu as pltpu
from jax.experimental.pallas import tpu_sc as plsc

N_NODES = 10000
IN_CH = 128
OUT_CH = 128

NC = 2
NS = 16
L = 16

NPAD = 10240
ROWS_PER_TILE = NPAD // NS
CHUNK = 128
CPW = 88
PPW = CPW // 2
NW = NC * NS
EPAD = NW * CPW * CHUNK
NREAL_PAIRS = 1290

_sc_mesh = plsc.VectorSubcoreMesh(
    core_axis_name="c", subcore_axis_name="s", num_cores=NC, num_subcores=NS
)


def _mm_body(x_ref, w_ref, o_ref):
    o_ref[...] = jnp.dot(x_ref[...], w_ref[...], preferred_element_type=jnp.float32)


def _matmul(xp, W):
    blk = 512
    return pl.pallas_call(
        _mm_body,
        grid=(NPAD // blk,),
        in_specs=[
            pl.BlockSpec((blk, IN_CH), lambda i: (i, 0)),
            pl.BlockSpec((IN_CH, OUT_CH), lambda i: (0, 0)),
        ],
        out_specs=pl.BlockSpec((blk, OUT_CH), lambda i: (i, 0)),
        out_shape=jax.ShapeDtypeStruct((NPAD, OUT_CH), jnp.float32),
    )(xp, W)


def _deg_body(dst_hbm, ew_hbm, deg_hbm, dstv, ewv, degsh, zb, sem):
    c = lax.axis_index("c")
    s = lax.axis_index("s")
    wid = c * NS + s
    base = wid * CPW
    pltpu.sync_copy(dst_hbm.at[pl.ds(base, CPW)], dstv)
    pltpu.sync_copy(ew_hbm.at[pl.ds(base, CPW)], ewv)

    def zloop(i, carry):
        zb[pl.ds(i * L, L)] = jnp.zeros((L,), jnp.float32)
        return carry

    lax.fori_loop(0, ROWS_PER_TILE // L, zloop, 0)
    pltpu.sync_copy(zb, degsh.at[pl.ds(s * ROWS_PER_TILE, ROWS_PER_TILE)])
    plsc.subcore_barrier()

    def sloop(j, carry):
        pltpu.async_copy(ewv.at[j], degsh.at[dstv.at[j]], sem, add=True)
        return carry

    lax.fori_loop(0, CPW, sloop, 0)

    def dloop(j, carry):
        pltpu.make_async_copy(ewv.at[j], degsh.at[dstv.at[j]], sem).wait()
        return carry

    lax.fori_loop(0, CPW, dloop, 0)
    plsc.subcore_barrier()
    pltpu.sync_copy(degsh.at[pl.ds(s * ROWS_PER_TILE, ROWS_PER_TILE)], zb)
    pltpu.sync_copy(zb, deg_hbm.at[pl.ds(c * NPAD + s * ROWS_PER_TILE, ROWS_PER_TILE)])


def _deg(dst2, ew2):
    return pl.kernel(
        _deg_body,
        out_type=jax.ShapeDtypeStruct((NC * NPAD,), jnp.float32),
        mesh=_sc_mesh,
        scratch_types=[
            pltpu.VMEM((CPW, CHUNK), jnp.int32),
            pltpu.VMEM((CPW, CHUNK), jnp.float32),
            pltpu.VMEM_SHARED((NPAD,), jnp.float32),
            pltpu.VMEM((ROWS_PER_TILE,), jnp.float32),
            pltpu.SemaphoreType.DMA,
        ],
    )(dst2, ew2)


def _dis_body(deg_ref, dis_ref):
    d = deg_ref[0] + deg_ref[1]
    dis_ref[...] = jnp.where(d > 0, lax.rsqrt(jnp.where(d > 0, d, 1.0)), 0.0)


def _dis(deg_part):
    return pl.pallas_call(
        _dis_body,
        out_shape=jax.ShapeDtypeStruct((NPAD // 128, 128), jnp.float32),
    )(deg_part)


def _scale_body(h_ref, dis_ref, o_ref):
    o_ref[...] = h_ref[...] * dis_ref[...]


def _scale(h, dis2):
    blk = 512
    return pl.pallas_call(
        _scale_body,
        grid=(NPAD // blk,),
        in_specs=[
            pl.BlockSpec((blk, OUT_CH), lambda i: (i, 0)),
            pl.BlockSpec((blk, 1), lambda i: (i, 0)),
        ],
        out_specs=pl.BlockSpec((blk, OUT_CH), lambda i: (i, 0)),
        out_shape=jax.ShapeDtypeStruct((NPAD, OUT_CH), jnp.float32),
    )(h, dis2)


def _row_scale(rowb, ewv, j):
    def scale(rg, carry):
        n16 = ewv[j, pl.ds(rg * L, L)]
        for i in range(L):
            nb = n16.at[jnp.full((L,), i, jnp.int32)].get(
                mode="promise_in_bounds")
            r = rg * L + i
            for q in range(OUT_CH // L):
                sl = pl.ds(q * L, L)
                rowb[r, sl] = rowb[r, sl] * nb
        return carry

    lax.fori_loop(0, CHUNK // L, scale, 0)


def _msg_body(src_hbm, dst_hbm, ew_hbm, hs_hbm, out_hbm,
              srcv, dstv, ewv, rowa, rowc, accsh, gsa, gsb, ssa, ssb):
    c = lax.axis_index("c")
    s = lax.axis_index("s")
    wid = c * NS + s
    base = wid * CPW
    pmax = (NREAL_PAIRS - 1 - wid) // NW + 1

    def zr(r, carry):
        for q in range(OUT_CH // L):
            rowa[r, pl.ds(q * L, L)] = jnp.zeros((L,), jnp.float32)
        return carry

    lax.fori_loop(0, CHUNK, zr, 0)
    for k in range(ROWS_PER_TILE // CHUNK):
        pltpu.sync_copy(rowa, accsh.at[pl.ds(s * ROWS_PER_TILE + k * CHUNK, CHUNK)])
    plsc.subcore_barrier()

    def pbody(k, carry):
        cb = base + 2 * k
        pltpu.sync_copy(src_hbm.at[pl.ds(cb, 2)], srcv)
        pltpu.sync_copy(dst_hbm.at[pl.ds(cb, 2)], dstv)
        pltpu.sync_copy(ew_hbm.at[pl.ds(cb, 2)], ewv)
        pltpu.async_copy(hs_hbm.at[srcv.at[0]], rowa, gsa)
        pltpu.async_copy(hs_hbm.at[srcv.at[1]], rowc, gsb)
        pltpu.make_async_copy(hs_hbm.at[srcv.at[0]], rowa, gsa).wait()
        _row_scale(rowa, ewv, 0)
        pltpu.async_copy(rowa, accsh.at[dstv.at[0]], ssa, add=True)
        pltpu.make_async_copy(hs_hbm.at[srcv.at[1]], rowc, gsb).wait()
        _row_scale(rowc, ewv, 1)
        pltpu.async_copy(rowc, accsh.at[dstv.at[1]], ssb, add=True)
        pltpu.make_async_copy(rowa, accsh.at[dstv.at[0]], ssa).wait()
        pltpu.make_async_copy(rowc, accsh.at[dstv.at[1]], ssb).wait()
        return carry

    lax.fori_loop(0, pmax, pbody, 0)
    plsc.subcore_barrier()
    for k in range(ROWS_PER_TILE // CHUNK):
        off = s * ROWS_PER_TILE + k * CHUNK
        pltpu.sync_copy(accsh.at[pl.ds(off, CHUNK)], rowa)
        pltpu.sync_copy(rowa, out_hbm.at[c, pl.ds(off, CHUNK)])


def _msg(src2, dst2, ew2, hs):
    return pl.kernel(
        _msg_body,
        out_type=jax.ShapeDtypeStruct((NC, NPAD, OUT_CH), jnp.float32),
        mesh=_sc_mesh,
        scratch_types=[
            pltpu.VMEM((2, CHUNK), jnp.int32),
            pltpu.VMEM((2, CHUNK), jnp.int32),
            pltpu.VMEM((2, CHUNK), jnp.float32),
            pltpu.VMEM((CHUNK, OUT_CH), jnp.float32),
            pltpu.VMEM((CHUNK, OUT_CH), jnp.float32),
            pltpu.VMEM_SHARED((NPAD, OUT_CH), jnp.float32),
            pltpu.SemaphoreType.DMA,
            pltpu.SemaphoreType.DMA,
            pltpu.SemaphoreType.DMA,
            pltpu.SemaphoreType.DMA,
        ],
    )(src2, dst2, ew2, hs)


def _comb_body(p0_ref, p1_ref, dis_ref, b_ref, o_ref):
    o_ref[...] = (p0_ref[...] + p1_ref[...]) * dis_ref[...] + b_ref[...]


def _combine(p0, p1, dis2, b2):
    blk = 512
    return pl.pallas_call(
        _comb_body,
        grid=(NPAD // blk,),
        in_specs=[
            pl.BlockSpec((blk, OUT_CH), lambda i: (i, 0)),
            pl.BlockSpec((blk, OUT_CH), lambda i: (i, 0)),
            pl.BlockSpec((blk, 1), lambda i: (i, 0)),
            pl.BlockSpec((1, OUT_CH), lambda i: (0, 0)),
        ],
        out_specs=pl.BlockSpec((blk, OUT_CH), lambda i: (i, 0)),
        out_shape=jax.ShapeDtypeStruct((NPAD, OUT_CH), jnp.float32),
    )(p0, p1, dis2, b2)


def kernel(x, edge_index, edge_weight, W, b):
    E = edge_index.shape[1]
    src = edge_index[0].astype(jnp.int32)
    dst = edge_index[1].astype(jnp.int32)
    ew = edge_weight.astype(jnp.float32)

    loop = jnp.arange(NPAD, dtype=jnp.int32)
    n_fill = EPAD - E - NPAD
    src_f = jnp.concatenate([src, loop, jnp.zeros((n_fill,), jnp.int32)])
    dst_f = jnp.concatenate([dst, loop, jnp.zeros((n_fill,), jnp.int32)])
    ew_f = jnp.concatenate(
        [ew, jnp.ones((NPAD,), jnp.float32), jnp.zeros((n_fill,), jnp.float32)]
    )
    def _permute(a):
        return (a.reshape(PPW, NW, 2 * CHUNK)
                 .transpose(1, 0, 2)
                 .reshape(EPAD // CHUNK, CHUNK))

    src2 = _permute(src_f)
    dst2 = _permute(dst_f)
    ew2 = _permute(ew_f)

    xp = jnp.pad(x.astype(jnp.float32), ((0, NPAD - N_NODES), (0, 0)))
    h = _matmul(xp, W.astype(jnp.float32))

    deg_part = _deg(dst2, ew2)
    dis = _dis(deg_part.reshape(NC, NPAD // 128, 128)).reshape(NPAD, 1)

    hs = _scale(h, dis)
    parts = _msg(src2, dst2, ew2, hs)
    out = _combine(parts[0], parts[1], dis,
                   b.astype(jnp.float32).reshape(1, OUT_CH))
    return out[:N_NODES]
